# baseline (speedup 1.0000x reference)
import functools

import jax
import jax.numpy as jnp
from jax import lax
from jax.experimental import pallas as pl
from jax.experimental.pallas import tpu as pltpu
from jax.experimental.pallas import tpu_sc as plsc

T = 2048
D = 1024
E = 8
F = 2048
K = 2
BT = 256
NT = 24
NROWS = NT * BT
NF = 2
FB = F // NF


def _routing_kernel(logits_ref, scale_ref, p0_ref, p1_ref, w0_ref, w1_ref,
                    te_ref):
    logits = logits_ref[...]
    scale = scale_ref[...]
    iota_e = lax.broadcasted_iota(jnp.int32, (T, E), 1)

    m1 = jnp.max(logits, axis=1, keepdims=True)
    id1 = jnp.min(jnp.where(logits == m1, iota_e, E), axis=1, keepdims=True)
    sel1 = iota_e == id1
    masked = jnp.where(sel1, -jnp.inf, logits)
    m2 = jnp.max(masked, axis=1, keepdims=True)
    id2 = jnp.min(jnp.where(masked == m2, iota_e, E), axis=1, keepdims=True)
    sel2 = iota_e == id2

    ex = jnp.exp(logits - m1)
    probs = ex / jnp.sum(ex, axis=1, keepdims=True)
    gate = jnp.where(sel1 | sel2, probs, 0.0)
    renorm = jnp.sum(gate, axis=1, keepdims=True)
    renorm = jnp.where(renorm > 0.0, renorm, 1.0)
    ps = probs * scale
    w0 = jnp.sum(jnp.where(sel1, ps, 0.0), axis=1, keepdims=True) / renorm
    w1 = jnp.sum(jnp.where(sel2, ps, 0.0), axis=1, keepdims=True) / renorm

    oh1 = sel1.astype(jnp.float32)
    oh2 = sel2.astype(jnp.float32)

    def _cumsum0(a):
        s = 1
        while s < T:
            shifted = jnp.concatenate(
                [jnp.zeros((s, E), jnp.float32), a[: T - s, :]], axis=0)
            a = a + shifted
            s *= 2
        return a

    cs1 = _cumsum0(oh1)
    cs2 = _cumsum0(oh2)
    tot1 = cs1[T - 1:T, :]
    n_e = tot1 + cs2[T - 1:T, :]

    r0 = jnp.sum(jnp.where(sel1, cs1, 0.0), axis=1, keepdims=True) - 1.0
    r1 = (jnp.sum(jnp.where(sel2, cs2 + tot1, 0.0), axis=1, keepdims=True)
          - 1.0)

    nt_e = jnp.ceil(n_e / BT)
    strict_lt = (lax.broadcasted_iota(jnp.int32, (E, E), 0)
                 < lax.broadcasted_iota(jnp.int32, (E, E), 1))
    excl = jnp.dot(nt_e, strict_lt.astype(jnp.float32),
                   preferred_element_type=jnp.float32)
    p_start = excl * BT

    pos0 = jnp.sum(jnp.where(sel1, p_start, 0.0), axis=1, keepdims=True) + r0
    pos1 = jnp.sum(jnp.where(sel2, p_start, 0.0), axis=1, keepdims=True) + r1
    p0_ref[...] = pos0.astype(jnp.int32)
    p1_ref[...] = pos1.astype(jnp.int32)
    w0_ref[...] = w0
    w1_ref[...] = w1

    tile_iota = lax.broadcasted_iota(jnp.int32, (32, E), 0)
    cnt = jnp.sum((tile_iota >= excl.astype(jnp.int32)).astype(jnp.int32),
                  axis=1, keepdims=True) - 1
    te_ref[...] = jnp.clip(cnt, 0, E - 1)


def _routing(router_logits, per_expert_scale, interpret=False):
    out_shapes = (
        jax.ShapeDtypeStruct((T, 1), jnp.int32),
        jax.ShapeDtypeStruct((T, 1), jnp.int32),
        jax.ShapeDtypeStruct((T, 1), jnp.float32),
        jax.ShapeDtypeStruct((T, 1), jnp.float32),
        jax.ShapeDtypeStruct((32, 1), jnp.int32),
    )
    return pl.pallas_call(
        _routing_kernel,
        out_shape=out_shapes,
        interpret=interpret,
    )(router_logits, per_expert_scale.reshape(1, E))


def _gelu_exact(x):
    return 0.5 * x * (1.0 + lax.erf(x * 0.7071067811865476))


def _ffn_kernel(te_ref, xs_ref, wg_ref, wu_ref, wd_ref, sw_ref, y_ref):
    x = xs_ref[...].astype(jnp.bfloat16)
    g = jnp.dot(x, wg_ref[0], preferred_element_type=jnp.float32)
    u = jnp.dot(x, wu_ref[0], preferred_element_type=jnp.float32)
    a = (_gelu_exact(g) * u).astype(jnp.bfloat16)
    y = jnp.dot(a, wd_ref[0], preferred_element_type=jnp.float32)
    y_ref[...] = y * sw_ref[...]


def _ffn(xs, w_gate, w_up, w_down, sorted_w, te, interpret=False):
    y = pl.pallas_call(
        _ffn_kernel,
        grid_spec=pltpu.PrefetchScalarGridSpec(
            num_scalar_prefetch=1,
            grid=(NT,),
            in_specs=[
                pl.BlockSpec((BT, D), lambda t, te: (t, 0)),
                pl.BlockSpec((1, D, F), lambda t, te: (te[t], 0, 0)),
                pl.BlockSpec((1, D, F), lambda t, te: (te[t], 0, 0)),
                pl.BlockSpec((1, F, D), lambda t, te: (te[t], 0, 0)),
                pl.BlockSpec((BT, 1), lambda t, te: (t, 0)),
            ],
            out_specs=pl.BlockSpec((BT, D), lambda t, te: (t, 0)),
        ),
        out_shape=jax.ShapeDtypeStruct((NROWS, D), jnp.float32),
        compiler_params=pltpu.CompilerParams(
            vmem_limit_bytes=120 * 1024 * 1024),
        interpret=interpret,
    )(te, xs, w_gate, w_up, w_down, sorted_w)
    return y


NC = 2
NS = 16
NW = NC * NS
ZCH = NROWS // NS
PCH = (T * K) // NS
RCH = NROWS // NW
GCH = 48
TCH = T // NW
CCH = 32


def _sc_dispatch_body(x_hbm, pos_hbm, wv_hbm, xs_hbm, sw_hbm,
                      stok_sh, sw_sh, posb, valb, wb, zb_i, zb_f,
                      idxb, rows0, rows1, swb, sem0, sem1):
    c = lax.axis_index("c")
    s = lax.axis_index("s")
    wid = s * NC + c

    zero_i = jnp.zeros((16,), jnp.int32)
    zero_f = jnp.zeros((16,), jnp.float32)
    for j in range(ZCH // 16):
        zb_i[pl.ds(j * 16, 16)] = zero_i
        zb_f[pl.ds(j * 16, 16)] = zero_f
    pltpu.sync_copy(zb_i, stok_sh.at[pl.ds(s * ZCH, ZCH)])
    pltpu.sync_copy(zb_f, sw_sh.at[pl.ds(s * ZCH, ZCH)])
    plsc.subcore_barrier()

    iota = lax.iota(jnp.int32, 16)
    for cc in range(PCH // 128):
        gbase = s * PCH + cc * 128
        pltpu.sync_copy(pos_hbm.at[pl.ds(gbase, 128)], posb)
        pltpu.sync_copy(wv_hbm.at[pl.ds(gbase, 128)], wb)
        for j in range(8):
            g = iota + (gbase + j * 16)
            tokv = jnp.where(g >= T, g - T, g)
            valb[pl.ds(j * 16, 16)] = tokv
        pltpu.sync_copy(valb, stok_sh.at[posb], add=True)
        pltpu.sync_copy(wb, sw_sh.at[posb], add=True)
    plsc.subcore_barrier()

    rbase = wid * RCH
    pltpu.sync_copy(stok_sh.at[pl.ds(rbase, RCH)], idxb)
    bufs = (rows0, rows1)
    sems = (sem0, sem1)
    nch = RCH // GCH
    cps = [None] * nch
    for cc in range(2):
        cps[cc] = pltpu.async_copy(
            x_hbm.at[idxb.at[pl.ds(cc * GCH, GCH)]], bufs[cc % 2],
            sems[cc % 2])
    for cc in range(nch):
        cps[cc].wait()
        pltpu.sync_copy(bufs[cc % 2], xs_hbm.at[pl.ds(rbase + cc * GCH, GCH)])
        nxt = cc + 2
        if nxt < nch:
            cps[nxt] = pltpu.async_copy(
                x_hbm.at[idxb.at[pl.ds(nxt * GCH, GCH)]], bufs[nxt % 2],
                sems[nxt % 2])
    pltpu.sync_copy(sw_sh.at[pl.ds(rbase, RCH)], swb)
    pltpu.sync_copy(swb, sw_hbm.at[pl.ds(rbase, RCH)])


def _sc_dispatch(x3, pos_all, w_all):
    mesh = plsc.VectorSubcoreMesh(core_axis_name="c", subcore_axis_name="s")
    f = pl.kernel(
        _sc_dispatch_body,
        out_type=(
            jax.ShapeDtypeStruct((NROWS, D), jnp.float32),
            jax.ShapeDtypeStruct((NROWS,), jnp.float32),
        ),
        mesh=mesh,
        scratch_types=[
            pltpu.VMEM_SHARED((NROWS,), jnp.int32),
            pltpu.VMEM_SHARED((NROWS,), jnp.float32),
            pltpu.VMEM((128,), jnp.int32),
            pltpu.VMEM((128,), jnp.int32),
            pltpu.VMEM((128,), jnp.float32),
            pltpu.VMEM((ZCH,), jnp.int32),
            pltpu.VMEM((ZCH,), jnp.float32),
            pltpu.VMEM((RCH,), jnp.int32),
            pltpu.VMEM((GCH, D), jnp.float32),
            pltpu.VMEM((GCH, D), jnp.float32),
            pltpu.VMEM((RCH,), jnp.float32),
            pltpu.SemaphoreType.DMA,
            pltpu.SemaphoreType.DMA,
        ],
    )
    return f(x3, pos_all, w_all)


def _sc_combine_body(y_hbm, p0_hbm, p1_hbm, out_hbm,
                     i0, i1, y0, y1, sem):
    c = lax.axis_index("c")
    s = lax.axis_index("s")
    wid = s * NC + c
    tb = wid * TCH
    for cc in range(TCH // CCH):
        pltpu.sync_copy(p0_hbm.at[pl.ds(tb + cc * CCH, CCH)], i0)
        pltpu.sync_copy(p1_hbm.at[pl.ds(tb + cc * CCH, CCH)], i1)
        pltpu.async_copy(y_hbm.at[i0], y0, sem).wait()
        pltpu.async_copy(y_hbm.at[i1], y1, sem).wait()

        def body(r, carry):
            for v in range(D // 16):
                sl = pl.ds(v * 16, 16)
                y0[r, sl] = y0[r, sl] + y1[r, sl]
            return carry

        lax.fori_loop(0, CCH, body, 0)
        pltpu.sync_copy(y0, out_hbm.at[pl.ds(tb + cc * CCH, CCH)])


def _sc_combine(y, p0, p1):
    mesh = plsc.VectorSubcoreMesh(core_axis_name="c", subcore_axis_name="s")
    f = pl.kernel(
        _sc_combine_body,
        out_type=jax.ShapeDtypeStruct((T, D), jnp.float32),
        mesh=mesh,
        scratch_types=[
            pltpu.VMEM((CCH,), jnp.int32),
            pltpu.VMEM((CCH,), jnp.int32),
            pltpu.VMEM((CCH, D), jnp.float32),
            pltpu.VMEM((CCH, D), jnp.float32),
            pltpu.SemaphoreType.DMA,
        ],
    )
    return f(y, p0, p1)


def kernel(x, router_logits, per_expert_scale, w_gate, w_up, w_down):
    p0, p1, w0, w1, te = _routing(router_logits, per_expert_scale)
    p0 = p0.reshape(T)
    p1 = p1.reshape(T)
    te = te.reshape(32)

    pos_all = jnp.concatenate([p0, p1])
    w_all = jnp.concatenate([w0.reshape(T), w1.reshape(T)])

    xs, sorted_w = _sc_dispatch(x, pos_all, w_all)
    y = _ffn(xs, w_gate.astype(jnp.bfloat16), w_up.astype(jnp.bfloat16),
             w_down.astype(jnp.bfloat16), sorted_w.reshape(NROWS, 1), te)
    out = _sc_combine(y, p0, p1)
    return out


# speedup vs baseline: 1.1867x; 1.1867x over previous
import functools

import jax
import jax.numpy as jnp
from jax import lax
from jax.experimental import pallas as pl
from jax.experimental.pallas import tpu as pltpu
from jax.experimental.pallas import tpu_sc as plsc

T = 2048
D = 1024
E = 8
F = 2048
K = 2
BT = 256
NT = 24
NROWS = NT * BT
NF = 2
FB = F // NF


def _routing_kernel(logits_ref, scale_ref, p0_ref, p1_ref, w0_ref, w1_ref,
                    te_ref):
    logits = logits_ref[...]
    scale = scale_ref[...]
    iota_e = lax.broadcasted_iota(jnp.int32, (T, E), 1)

    m1 = jnp.max(logits, axis=1, keepdims=True)
    id1 = jnp.min(jnp.where(logits == m1, iota_e, E), axis=1, keepdims=True)
    sel1 = iota_e == id1
    masked = jnp.where(sel1, -jnp.inf, logits)
    m2 = jnp.max(masked, axis=1, keepdims=True)
    id2 = jnp.min(jnp.where(masked == m2, iota_e, E), axis=1, keepdims=True)
    sel2 = iota_e == id2

    ex = jnp.exp(logits - m1)
    probs = ex / jnp.sum(ex, axis=1, keepdims=True)
    gate = jnp.where(sel1 | sel2, probs, 0.0)
    renorm = jnp.sum(gate, axis=1, keepdims=True)
    renorm = jnp.where(renorm > 0.0, renorm, 1.0)
    ps = probs * scale
    w0 = jnp.sum(jnp.where(sel1, ps, 0.0), axis=1, keepdims=True) / renorm
    w1 = jnp.sum(jnp.where(sel2, ps, 0.0), axis=1, keepdims=True) / renorm

    oh1 = sel1.astype(jnp.float32)
    oh2 = sel2.astype(jnp.float32)

    def _cumsum0(a):
        s = 1
        while s < T:
            shifted = jnp.concatenate(
                [jnp.zeros((s, E), jnp.float32), a[: T - s, :]], axis=0)
            a = a + shifted
            s *= 2
        return a

    cs1 = _cumsum0(oh1)
    cs2 = _cumsum0(oh2)
    tot1 = cs1[T - 1:T, :]
    n_e = tot1 + cs2[T - 1:T, :]

    r0 = jnp.sum(jnp.where(sel1, cs1, 0.0), axis=1, keepdims=True) - 1.0
    r1 = (jnp.sum(jnp.where(sel2, cs2 + tot1, 0.0), axis=1, keepdims=True)
          - 1.0)

    nt_e = jnp.ceil(n_e / BT)
    strict_lt = (lax.broadcasted_iota(jnp.int32, (E, E), 0)
                 < lax.broadcasted_iota(jnp.int32, (E, E), 1))
    excl = jnp.dot(nt_e, strict_lt.astype(jnp.float32),
                   preferred_element_type=jnp.float32)
    p_start = excl * BT

    pos0 = jnp.sum(jnp.where(sel1, p_start, 0.0), axis=1, keepdims=True) + r0
    pos1 = jnp.sum(jnp.where(sel2, p_start, 0.0), axis=1, keepdims=True) + r1
    p0_ref[...] = pos0.astype(jnp.int32)
    p1_ref[...] = pos1.astype(jnp.int32)
    w0_ref[...] = w0
    w1_ref[...] = w1

    tile_iota = lax.broadcasted_iota(jnp.int32, (32, E), 0)
    cnt = jnp.sum((tile_iota >= excl.astype(jnp.int32)).astype(jnp.int32),
                  axis=1, keepdims=True) - 1
    te_ref[...] = jnp.clip(cnt, 0, E - 1)


def _routing(router_logits, per_expert_scale, interpret=False):
    out_shapes = (
        jax.ShapeDtypeStruct((T, 1), jnp.int32),
        jax.ShapeDtypeStruct((T, 1), jnp.int32),
        jax.ShapeDtypeStruct((T, 1), jnp.float32),
        jax.ShapeDtypeStruct((T, 1), jnp.float32),
        jax.ShapeDtypeStruct((32, 1), jnp.int32),
    )
    return pl.pallas_call(
        _routing_kernel,
        out_shape=out_shapes,
        interpret=interpret,
    )(router_logits, per_expert_scale.reshape(1, E))


def _gelu_exact(x):
    return 0.5 * x * (1.0 + lax.erf(x * 0.7071067811865476))


def _ffn_kernel(te_ref, xs_ref, wg_ref, wu_ref, wd_ref, sw_ref, y_ref):
    x = xs_ref[...]
    g = jnp.dot(x, wg_ref[0], preferred_element_type=jnp.float32)
    u = jnp.dot(x, wu_ref[0], preferred_element_type=jnp.float32)
    a = _gelu_exact(g) * u
    y = jnp.dot(a, wd_ref[0], preferred_element_type=jnp.float32)
    y_ref[...] = y * sw_ref[...]


def _ffn(xs, w_gate, w_up, w_down, sorted_w, te, interpret=False):
    y = pl.pallas_call(
        _ffn_kernel,
        grid_spec=pltpu.PrefetchScalarGridSpec(
            num_scalar_prefetch=1,
            grid=(NT,),
            in_specs=[
                pl.BlockSpec((BT, D), lambda t, te: (t, 0)),
                pl.BlockSpec((1, D, F), lambda t, te: (te[t], 0, 0)),
                pl.BlockSpec((1, D, F), lambda t, te: (te[t], 0, 0)),
                pl.BlockSpec((1, F, D), lambda t, te: (te[t], 0, 0)),
                pl.BlockSpec((BT, 1), lambda t, te: (t, 0)),
            ],
            out_specs=pl.BlockSpec((BT, D), lambda t, te: (t, 0)),
        ),
        out_shape=jax.ShapeDtypeStruct((NROWS, D), jnp.float32),
        compiler_params=pltpu.CompilerParams(
            vmem_limit_bytes=120 * 1024 * 1024),
        interpret=interpret,
    )(te, xs, w_gate, w_up, w_down, sorted_w)
    return y


NC = 2
NS = 16
NW = NC * NS
ZCH = NROWS // NS
PCH = (T * K) // NS
RCH = NROWS // NW
GCH = 48
TCH = T // NW
CCH = 32


def _sc_sort_body(pos_hbm, wv_hbm, stok_hbm, sw_hbm,
                  stok_sh, sw_sh, posb, valb, wb, zb_i, zb_f, outb_i, outb_f):
    c = lax.axis_index("c")
    s = lax.axis_index("s")
    wid = s * NC + c

    zero_i = jnp.zeros((16,), jnp.int32)
    zero_f = jnp.zeros((16,), jnp.float32)
    for j in range(ZCH // 16):
        zb_i[pl.ds(j * 16, 16)] = zero_i
        zb_f[pl.ds(j * 16, 16)] = zero_f
    pltpu.sync_copy(zb_i, stok_sh.at[pl.ds(s * ZCH, ZCH)])
    pltpu.sync_copy(zb_f, sw_sh.at[pl.ds(s * ZCH, ZCH)])
    plsc.subcore_barrier()

    iota = lax.iota(jnp.int32, 16)
    for cc in range(PCH // 128):
        gbase = s * PCH + cc * 128
        pltpu.sync_copy(pos_hbm.at[pl.ds(gbase, 128)], posb)
        pltpu.sync_copy(wv_hbm.at[pl.ds(gbase, 128)], wb)
        for j in range(8):
            g = iota + (gbase + j * 16)
            tokv = jnp.where(g >= T, g - T, g)
            valb[pl.ds(j * 16, 16)] = tokv
        pltpu.sync_copy(valb, stok_sh.at[posb], add=True)
        pltpu.sync_copy(wb, sw_sh.at[posb], add=True)
    plsc.subcore_barrier()

    rbase = wid * RCH
    pltpu.sync_copy(stok_sh.at[pl.ds(rbase, RCH)], outb_i)
    pltpu.sync_copy(outb_i, stok_hbm.at[pl.ds(rbase, RCH)])
    pltpu.sync_copy(sw_sh.at[pl.ds(rbase, RCH)], outb_f)
    pltpu.sync_copy(outb_f, sw_hbm.at[pl.ds(rbase, RCH)])


def _sc_sort(pos_all, w_all):
    mesh = plsc.VectorSubcoreMesh(core_axis_name="c", subcore_axis_name="s")
    f = pl.kernel(
        _sc_sort_body,
        out_type=(
            jax.ShapeDtypeStruct((NROWS,), jnp.int32),
            jax.ShapeDtypeStruct((NROWS,), jnp.float32),
        ),
        mesh=mesh,
        scratch_types=[
            pltpu.VMEM_SHARED((NROWS,), jnp.int32),
            pltpu.VMEM_SHARED((NROWS,), jnp.float32),
            pltpu.VMEM((128,), jnp.int32),
            pltpu.VMEM((128,), jnp.int32),
            pltpu.VMEM((128,), jnp.float32),
            pltpu.VMEM((ZCH,), jnp.int32),
            pltpu.VMEM((ZCH,), jnp.float32),
            pltpu.VMEM((RCH,), jnp.int32),
            pltpu.VMEM((RCH,), jnp.float32),
        ],
    )
    return f(pos_all, w_all)


def _sc_gather_body(x_hbm, stok_hbm, xs_hbm, idxb, rows0, rows1, sem0, sem1):
    c = lax.axis_index("c")
    s = lax.axis_index("s")
    wid = s * NC + c
    rbase = wid * RCH
    pltpu.sync_copy(stok_hbm.at[pl.ds(rbase, RCH)], idxb)
    bufs = (rows0, rows1)
    sems = (sem0, sem1)
    nch = RCH // GCH
    cps = [None] * nch
    for cc in range(2):
        cps[cc] = pltpu.async_copy(
            x_hbm.at[idxb.at[pl.ds(cc * GCH, GCH)]], bufs[cc % 2],
            sems[cc % 2])
    for cc in range(nch):
        cps[cc].wait()
        pltpu.sync_copy(bufs[cc % 2], xs_hbm.at[pl.ds(rbase + cc * GCH, GCH)])
        nxt = cc + 2
        if nxt < nch:
            cps[nxt] = pltpu.async_copy(
                x_hbm.at[idxb.at[pl.ds(nxt * GCH, GCH)]], bufs[nxt % 2],
                sems[nxt % 2])


def _sc_gather(x, stok):
    mesh = plsc.VectorSubcoreMesh(core_axis_name="c", subcore_axis_name="s")
    f = pl.kernel(
        _sc_gather_body,
        out_type=jax.ShapeDtypeStruct((NROWS, D), jnp.float32),
        mesh=mesh,
        scratch_types=[
            pltpu.VMEM((RCH,), jnp.int32),
            pltpu.VMEM((GCH, D), jnp.float32),
            pltpu.VMEM((GCH, D), jnp.float32),
            pltpu.SemaphoreType.DMA,
            pltpu.SemaphoreType.DMA,
        ],
    )
    return f(x, stok)


def _sc_dispatch(x, pos_all, w_all):
    stok, sw = _sc_sort(pos_all, w_all)
    xs = _sc_gather(x, stok)
    return xs, sw


def _sc_combine_body(y_hbm, p0_hbm, p1_hbm, out_hbm,
                     i0, i1, y0, y1, sem):
    c = lax.axis_index("c")
    s = lax.axis_index("s")
    wid = s * NC + c
    tb = wid * TCH
    for cc in range(TCH // CCH):
        pltpu.sync_copy(p0_hbm.at[pl.ds(tb + cc * CCH, CCH)], i0)
        pltpu.sync_copy(p1_hbm.at[pl.ds(tb + cc * CCH, CCH)], i1)
        pltpu.async_copy(y_hbm.at[i0], y0, sem).wait()
        pltpu.async_copy(y_hbm.at[i1], y1, sem).wait()

        def body(r, carry):
            for v in range(D // 16):
                sl = pl.ds(v * 16, 16)
                y0[r, sl] = y0[r, sl] + y1[r, sl]
            return carry

        lax.fori_loop(0, CCH, body, 0)
        pltpu.sync_copy(y0, out_hbm.at[pl.ds(tb + cc * CCH, CCH)])


def _sc_combine(y, p0, p1):
    mesh = plsc.VectorSubcoreMesh(core_axis_name="c", subcore_axis_name="s")
    f = pl.kernel(
        _sc_combine_body,
        out_type=jax.ShapeDtypeStruct((T, D), jnp.float32),
        mesh=mesh,
        scratch_types=[
            pltpu.VMEM((CCH,), jnp.int32),
            pltpu.VMEM((CCH,), jnp.int32),
            pltpu.VMEM((CCH, D), jnp.float32),
            pltpu.VMEM((CCH, D), jnp.float32),
            pltpu.SemaphoreType.DMA,
        ],
    )
    return f(y, p0, p1)


def kernel(x, router_logits, per_expert_scale, w_gate, w_up, w_down):
    p0, p1, w0, w1, te = _routing(router_logits, per_expert_scale)
    p0 = p0.reshape(T)
    p1 = p1.reshape(T)
    te = te.reshape(32)

    pos_all = jnp.concatenate([p0, p1])
    w_all = jnp.concatenate([w0.reshape(T), w1.reshape(T)])

    xs, sorted_w = _sc_dispatch(x, pos_all, w_all)
    y = _ffn(xs, w_gate, w_up, w_down, sorted_w.reshape(NROWS, 1), te)
    out = _sc_combine(y, p0, p1)
    return out


# speedup vs baseline: 1.3769x; 1.1603x over previous
import functools

import jax
import jax.numpy as jnp
from jax import lax
from jax.experimental import pallas as pl
from jax.experimental.pallas import tpu as pltpu
from jax.experimental.pallas import tpu_sc as plsc

T = 2048
D = 1024
E = 8
F = 2048
K = 2
BT = 128
NT = 40
NROWS = NT * BT
NF = 2
FB = F // NF


def _routing_kernel(logits_ref, scale_ref, p0_ref, p1_ref, w0_ref, w1_ref,
                    te_ref):
    logits = logits_ref[...]
    scale = scale_ref[...]
    iota_e = lax.broadcasted_iota(jnp.int32, (T, E), 1)

    m1 = jnp.max(logits, axis=1, keepdims=True)
    id1 = jnp.min(jnp.where(logits == m1, iota_e, E), axis=1, keepdims=True)
    sel1 = iota_e == id1
    masked = jnp.where(sel1, -jnp.inf, logits)
    m2 = jnp.max(masked, axis=1, keepdims=True)
    id2 = jnp.min(jnp.where(masked == m2, iota_e, E), axis=1, keepdims=True)
    sel2 = iota_e == id2

    ex = jnp.exp(logits - m1)
    probs = ex / jnp.sum(ex, axis=1, keepdims=True)
    gate = jnp.where(sel1 | sel2, probs, 0.0)
    renorm = jnp.sum(gate, axis=1, keepdims=True)
    renorm = jnp.where(renorm > 0.0, renorm, 1.0)
    ps = probs * scale
    w0 = jnp.sum(jnp.where(sel1, ps, 0.0), axis=1, keepdims=True) / renorm
    w1 = jnp.sum(jnp.where(sel2, ps, 0.0), axis=1, keepdims=True) / renorm

    oh1 = sel1.astype(jnp.float32)
    oh2 = sel2.astype(jnp.float32)

    def _cumsum0(a):
        s = 1
        while s < T:
            shifted = jnp.concatenate(
                [jnp.zeros((s, E), jnp.float32), a[: T - s, :]], axis=0)
            a = a + shifted
            s *= 2
        return a

    cs1 = _cumsum0(oh1)
    cs2 = _cumsum0(oh2)
    tot1 = cs1[T - 1:T, :]
    n_e = tot1 + cs2[T - 1:T, :]

    r0 = jnp.sum(jnp.where(sel1, cs1, 0.0), axis=1, keepdims=True) - 1.0
    r1 = (jnp.sum(jnp.where(sel2, cs2 + tot1, 0.0), axis=1, keepdims=True)
          - 1.0)

    nt_e = jnp.ceil(n_e / BT)
    strict_lt = (lax.broadcasted_iota(jnp.int32, (E, E), 0)
                 < lax.broadcasted_iota(jnp.int32, (E, E), 1))
    excl = jnp.dot(nt_e, strict_lt.astype(jnp.float32),
                   preferred_element_type=jnp.float32)
    p_start = excl * BT

    pos0 = jnp.sum(jnp.where(sel1, p_start, 0.0), axis=1, keepdims=True) + r0
    pos1 = jnp.sum(jnp.where(sel2, p_start, 0.0), axis=1, keepdims=True) + r1
    p0_ref[...] = pos0.astype(jnp.int32)
    p1_ref[...] = pos1.astype(jnp.int32)
    w0_ref[...] = w0
    w1_ref[...] = w1

    tile_iota = lax.broadcasted_iota(jnp.int32, (NT, E), 0)
    cnt = jnp.sum((tile_iota >= excl.astype(jnp.int32)).astype(jnp.int32),
                  axis=1, keepdims=True) - 1
    te_ref[...] = jnp.clip(cnt, 0, E - 1)


def _routing(router_logits, per_expert_scale, interpret=False):
    out_shapes = (
        jax.ShapeDtypeStruct((T, 1), jnp.int32),
        jax.ShapeDtypeStruct((T, 1), jnp.int32),
        jax.ShapeDtypeStruct((T, 1), jnp.float32),
        jax.ShapeDtypeStruct((T, 1), jnp.float32),
        jax.ShapeDtypeStruct((NT, 1), jnp.int32),
    )
    return pl.pallas_call(
        _routing_kernel,
        out_shape=out_shapes,
        interpret=interpret,
    )(router_logits, per_expert_scale.reshape(1, E))


def _gelu_exact(x):
    return 0.5 * x * (1.0 + lax.erf(x * 0.7071067811865476))


def _ffn_kernel(te_ref, xs_ref, wg_ref, wu_ref, wd_ref, sw_ref, y_ref):
    x = xs_ref[...]
    g = jnp.dot(x, wg_ref[0], preferred_element_type=jnp.float32)
    u = jnp.dot(x, wu_ref[0], preferred_element_type=jnp.float32)
    a = _gelu_exact(g) * u
    y = jnp.dot(a, wd_ref[0], preferred_element_type=jnp.float32)
    y_ref[...] = y * sw_ref[...]


def _ffn(xs, w_gate, w_up, w_down, sorted_w, te, interpret=False):
    y = pl.pallas_call(
        _ffn_kernel,
        grid_spec=pltpu.PrefetchScalarGridSpec(
            num_scalar_prefetch=1,
            grid=(NT,),
            in_specs=[
                pl.BlockSpec((BT, D), lambda t, te: (t, 0)),
                pl.BlockSpec((1, D, F), lambda t, te: (te[t], 0, 0)),
                pl.BlockSpec((1, D, F), lambda t, te: (te[t], 0, 0)),
                pl.BlockSpec((1, F, D), lambda t, te: (te[t], 0, 0)),
                pl.BlockSpec((BT, 1), lambda t, te: (t, 0)),
            ],
            out_specs=pl.BlockSpec((BT, D), lambda t, te: (t, 0)),
        ),
        out_shape=jax.ShapeDtypeStruct((NROWS, D), jnp.float32),
        compiler_params=pltpu.CompilerParams(
            vmem_limit_bytes=120 * 1024 * 1024),
        interpret=interpret,
    )(te, xs, w_gate, w_up, w_down, sorted_w)
    return y


NC = 2
NS = 16
NW = NC * NS
ZCH = NROWS // NS
PCH = (T * K) // NS
RCH = NROWS // NW
GCH = 40
TCH = T // NW
CCH = 32


def _sc_sort_body(pos_hbm, wv_hbm, stok_hbm, sw_hbm,
                  stok_sh, sw_sh, posb, valb, wb, zb_i, zb_f, outb_i, outb_f):
    c = lax.axis_index("c")
    s = lax.axis_index("s")
    wid = s * NC + c

    zero_i = jnp.zeros((16,), jnp.int32)
    zero_f = jnp.zeros((16,), jnp.float32)
    for j in range(ZCH // 16):
        zb_i[pl.ds(j * 16, 16)] = zero_i
        zb_f[pl.ds(j * 16, 16)] = zero_f
    pltpu.sync_copy(zb_i, stok_sh.at[pl.ds(s * ZCH, ZCH)])
    pltpu.sync_copy(zb_f, sw_sh.at[pl.ds(s * ZCH, ZCH)])
    plsc.subcore_barrier()

    iota = lax.iota(jnp.int32, 16)
    for cc in range(PCH // 128):
        gbase = s * PCH + cc * 128
        pltpu.sync_copy(pos_hbm.at[pl.ds(gbase, 128)], posb)
        pltpu.sync_copy(wv_hbm.at[pl.ds(gbase, 128)], wb)
        for j in range(8):
            g = iota + (gbase + j * 16)
            tokv = jnp.where(g >= T, g - T, g)
            valb[pl.ds(j * 16, 16)] = tokv
        pltpu.sync_copy(valb, stok_sh.at[posb], add=True)
        pltpu.sync_copy(wb, sw_sh.at[posb], add=True)
    plsc.subcore_barrier()

    rbase = wid * RCH
    pltpu.sync_copy(stok_sh.at[pl.ds(rbase, RCH)], outb_i)
    pltpu.sync_copy(outb_i, stok_hbm.at[pl.ds(rbase, RCH)])
    pltpu.sync_copy(sw_sh.at[pl.ds(rbase, RCH)], outb_f)
    pltpu.sync_copy(outb_f, sw_hbm.at[pl.ds(rbase, RCH)])


def _sc_sort(pos_all, w_all):
    mesh = plsc.VectorSubcoreMesh(core_axis_name="c", subcore_axis_name="s")
    f = pl.kernel(
        _sc_sort_body,
        out_type=(
            jax.ShapeDtypeStruct((NROWS,), jnp.int32),
            jax.ShapeDtypeStruct((NROWS,), jnp.float32),
        ),
        mesh=mesh,
        scratch_types=[
            pltpu.VMEM_SHARED((NROWS,), jnp.int32),
            pltpu.VMEM_SHARED((NROWS,), jnp.float32),
            pltpu.VMEM((128,), jnp.int32),
            pltpu.VMEM((128,), jnp.int32),
            pltpu.VMEM((128,), jnp.float32),
            pltpu.VMEM((ZCH,), jnp.int32),
            pltpu.VMEM((ZCH,), jnp.float32),
            pltpu.VMEM((RCH,), jnp.int32),
            pltpu.VMEM((RCH,), jnp.float32),
        ],
    )
    return f(pos_all, w_all)


def _sc_gather_body(x_hbm, stok_hbm, xs_hbm, idxb, rows0, rows1, sem0, sem1):
    c = lax.axis_index("c")
    s = lax.axis_index("s")
    wid = s * NC + c
    rbase = wid * RCH
    pltpu.sync_copy(stok_hbm.at[pl.ds(rbase, RCH)], idxb)
    bufs = (rows0, rows1)
    sems = (sem0, sem1)
    nch = RCH // GCH
    cps = [None] * nch
    for cc in range(2):
        cps[cc] = pltpu.async_copy(
            x_hbm.at[idxb.at[pl.ds(cc * GCH, GCH)]], bufs[cc % 2],
            sems[cc % 2])
    for cc in range(nch):
        cps[cc].wait()
        pltpu.sync_copy(bufs[cc % 2], xs_hbm.at[pl.ds(rbase + cc * GCH, GCH)])
        nxt = cc + 2
        if nxt < nch:
            cps[nxt] = pltpu.async_copy(
                x_hbm.at[idxb.at[pl.ds(nxt * GCH, GCH)]], bufs[nxt % 2],
                sems[nxt % 2])


def _sc_gather(x, stok):
    mesh = plsc.VectorSubcoreMesh(core_axis_name="c", subcore_axis_name="s")
    f = pl.kernel(
        _sc_gather_body,
        out_type=jax.ShapeDtypeStruct((NROWS, D), jnp.float32),
        mesh=mesh,
        scratch_types=[
            pltpu.VMEM((RCH,), jnp.int32),
            pltpu.VMEM((GCH, D), jnp.float32),
            pltpu.VMEM((GCH, D), jnp.float32),
            pltpu.SemaphoreType.DMA,
            pltpu.SemaphoreType.DMA,
        ],
    )
    return f(x, stok)


def _sc_dispatch(x, pos_all, w_all):
    stok, sw = _sc_sort(pos_all, w_all)
    xs = _sc_gather(x, stok)
    return xs, sw


def _sc_combine_body(y_hbm, p0_hbm, p1_hbm, out_hbm,
                     i0, i1, y0, y1, sem):
    c = lax.axis_index("c")
    s = lax.axis_index("s")
    wid = s * NC + c
    tb = wid * TCH
    for cc in range(TCH // CCH):
        pltpu.sync_copy(p0_hbm.at[pl.ds(tb + cc * CCH, CCH)], i0)
        pltpu.sync_copy(p1_hbm.at[pl.ds(tb + cc * CCH, CCH)], i1)
        pltpu.async_copy(y_hbm.at[i0], y0, sem).wait()
        pltpu.async_copy(y_hbm.at[i1], y1, sem).wait()

        def body(r, carry):
            for v in range(D // 16):
                sl = pl.ds(v * 16, 16)
                y0[r, sl] = y0[r, sl] + y1[r, sl]
            return carry

        lax.fori_loop(0, CCH, body, 0)
        pltpu.sync_copy(y0, out_hbm.at[pl.ds(tb + cc * CCH, CCH)])


def _sc_combine(y, p0, p1):
    mesh = plsc.VectorSubcoreMesh(core_axis_name="c", subcore_axis_name="s")
    f = pl.kernel(
        _sc_combine_body,
        out_type=jax.ShapeDtypeStruct((T, D), jnp.float32),
        mesh=mesh,
        scratch_types=[
            pltpu.VMEM((CCH,), jnp.int32),
            pltpu.VMEM((CCH,), jnp.int32),
            pltpu.VMEM((CCH, D), jnp.float32),
            pltpu.VMEM((CCH, D), jnp.float32),
            pltpu.SemaphoreType.DMA,
        ],
    )
    return f(y, p0, p1)


def kernel(x, router_logits, per_expert_scale, w_gate, w_up, w_down):
    p0, p1, w0, w1, te = _routing(router_logits, per_expert_scale)
    p0 = p0.reshape(T)
    p1 = p1.reshape(T)
    te = te.reshape(NT)

    pos_all = jnp.concatenate([p0, p1])
    w_all = jnp.concatenate([w0.reshape(T), w1.reshape(T)])

    xs, sorted_w = _sc_dispatch(x, pos_all, w_all)
    y = _ffn(xs, w_gate, w_up, w_down, sorted_w.reshape(NROWS, 1), te)
    out = _sc_combine(y, p0, p1)
    return out
